# baseline (speedup 1.0000x reference)
import functools

import jax
import jax.numpy as jnp
from jax.experimental import pallas as pl
from jax.experimental.pallas import tpu as pltpu

N = 8192
PAD_N = 10000
ALPHA = 0.1
NQ = 4
BQ = N // NQ


def _prologue_kernel(x_ref, w_ref, al_ref, ar_ref, hv_ref,
                     aa_ref, ab_ref, ba_ref, bb_ref):
    h = jnp.dot(x_ref[...], w_ref[...], preferred_element_type=jnp.float32)
    hv_ref[...] = h
    f1 = jnp.dot(h, al_ref[...], preferred_element_type=jnp.float32)
    f2 = jnp.dot(h, ar_ref[...], preferred_element_type=jnp.float32)
    aa_ref[...] = jnp.exp(-f1)
    ab_ref[...] = jnp.exp(-ALPHA * f1)
    ba_ref[...] = jnp.exp(-f2)
    bb_ref[...] = jnp.exp(-ALPHA * f2)


def _prologue(x, W, a):
    d = W.shape[1]
    al = a[0, :d].reshape(d, 1)
    ar = a[0, d:].reshape(d, 1)
    col = jax.ShapeDtypeStruct((N, 1), jnp.float32)
    hv, A, Aa, B, Ba = pl.pallas_call(
        _prologue_kernel,
        out_shape=(jax.ShapeDtypeStruct((N, d), jnp.float32),
                   col, col, col, col),
    )(x, W, al, ar)
    return hv, A, Aa, B.reshape(1, N), Ba.reshape(1, N)


def _gat_kernel(bi, d, fold_rs, A_ref, Aa_ref, B_ref, Ba_ref, hv_ref,
                aj0_ref, aj1_ref, aj2_ref, aj3_ref, out_ref):
    i = pl.program_id(0)
    adj_refs = (aj0_ref, aj1_ref, aj2_ref, aj3_ref)

    a1 = A_ref[pl.ds(i * bi, bi), :].astype(jnp.bfloat16)
    a2 = Aa_ref[pl.ds(i * bi, bi), :].astype(jnp.bfloat16)

    da = hv_ref.shape[1]
    acc = jnp.zeros((bi, da), jnp.float32)
    rs = jnp.zeros((bi, 1), jnp.float32)
    for q in range(NQ):
        b1 = B_ref[:, q * BQ:(q + 1) * BQ].astype(jnp.bfloat16)
        b2 = Ba_ref[:, q * BQ:(q + 1) * BQ].astype(jnp.bfloat16)
        e = jnp.minimum(a1 * b1, a2 * b2)
        e = jnp.where(adj_refs[q][...] > 0, e, jnp.bfloat16(0))
        hv = hv_ref[q * BQ:(q + 1) * BQ, :]
        acc = acc + jnp.dot(e, hv, preferred_element_type=jnp.float32)
        if not fold_rs:
            rs = rs + jnp.sum(e.astype(jnp.float32), axis=1, keepdims=True)

    if fold_rs:
        rs = acc[:, d:d + 1]
    hp = acc[:, :d] / (rs + 1e-10)
    out_ref[...] = jnp.maximum(hp, 0.0)


def _gat_layer(adj, hv_b16, d, A, Aa, B, Ba, bi, fold_rs):
    ni = N // bi
    da = hv_b16.shape[1]
    adj_specs = tuple(
        pl.BlockSpec((bi, BQ), functools.partial(lambda q, i: (i, q), q))
        for q in range(NQ))
    return pl.pallas_call(
        functools.partial(_gat_kernel, bi, d, fold_rs),
        grid=(ni,),
        in_specs=(
            pl.BlockSpec((N, 1), lambda i: (0, 0)),
            pl.BlockSpec((N, 1), lambda i: (0, 0)),
            pl.BlockSpec((1, N), lambda i: (0, 0)),
            pl.BlockSpec((1, N), lambda i: (0, 0)),
            pl.BlockSpec((N, da), lambda i: (0, 0)),
        ) + adj_specs,
        out_specs=pl.BlockSpec((bi, d), lambda i: (i, 0)),
        out_shape=jax.ShapeDtypeStruct((N, d), jnp.float32),
        compiler_params=pltpu.CompilerParams(
            dimension_semantics=("arbitrary",)),
    )(A, Aa, B, Ba, hv_b16, adj, adj, adj, adj)


def _pad_kernel(h2_ref, out_ref):
    out_ref[pl.ds(0, N), :] = h2_ref[...]
    out_ref[pl.ds(N, PAD_N - N), :] = jnp.zeros((PAD_N - N, h2_ref.shape[1]),
                                                jnp.float32)


def _pad_output(h2):
    d = h2.shape[1]
    return pl.pallas_call(
        _pad_kernel,
        out_shape=jax.ShapeDtypeStruct((PAD_N, d), jnp.float32),
    )(h2)


def kernel(x, adj, pad_n, pos_idx, W1, a1, W2, a2):
    hv1, A1, Aa1, B1, Ba1 = _prologue(x, W1, a1)
    h1 = _gat_layer(adj, hv1.astype(jnp.bfloat16), hv1.shape[1],
                    A1, Aa1, B1, Ba1, bi=512, fold_rs=False)
    hv2, A2, Aa2, B2, Ba2 = _prologue(h1, W2, a2)
    d2 = hv2.shape[1]
    hv2_aug = jnp.concatenate(
        [hv2, jnp.ones((N, 1), jnp.float32),
         jnp.zeros((N, 127 - d2), jnp.float32)], axis=1).astype(jnp.bfloat16)
    h2 = _gat_layer(adj, hv2_aug, d2, A2, Aa2, B2, Ba2, bi=512, fold_rs=True)
    return _pad_output(h2)


# speedup vs baseline: 1.0219x; 1.0219x over previous
import functools

import jax
import jax.numpy as jnp
from jax.experimental import pallas as pl
from jax.experimental.pallas import tpu as pltpu

N = 8192
PAD_N = 10000
ALPHA = 0.1
NQ = 8
BQ = N // NQ


def _prologue_kernel(x_ref, w_ref, al_ref, ar_ref, hv_ref,
                     aa_ref, ab_ref, ba_ref, bb_ref):
    h = jnp.dot(x_ref[...], w_ref[...], preferred_element_type=jnp.float32)
    hv_ref[...] = h
    f1 = jnp.dot(h, al_ref[...], preferred_element_type=jnp.float32)
    f2 = jnp.dot(h, ar_ref[...], preferred_element_type=jnp.float32)
    aa_ref[...] = jnp.exp(-f1)
    ab_ref[...] = jnp.exp(-ALPHA * f1)
    ba_ref[...] = jnp.exp(-f2)
    bb_ref[...] = jnp.exp(-ALPHA * f2)


def _prologue(x, W, a):
    d = W.shape[1]
    al = a[0, :d].reshape(d, 1)
    ar = a[0, d:].reshape(d, 1)
    col = jax.ShapeDtypeStruct((N, 1), jnp.float32)
    hv, A, Aa, B, Ba = pl.pallas_call(
        _prologue_kernel,
        out_shape=(jax.ShapeDtypeStruct((N, d), jnp.float32),
                   col, col, col, col),
    )(x, W, al, ar)
    return hv, A, Aa, B.reshape(1, N), Ba.reshape(1, N)


def _gat_kernel(bi, d, fold_rs, A_ref, Aa_ref, B_ref, Ba_ref, hv_ref,
                aj0_ref, aj1_ref, aj2_ref, aj3_ref,
                aj4_ref, aj5_ref, aj6_ref, aj7_ref, out_ref):
    i = pl.program_id(0)
    adj_refs = (aj0_ref, aj1_ref, aj2_ref, aj3_ref,
                aj4_ref, aj5_ref, aj6_ref, aj7_ref)

    a1 = A_ref[pl.ds(i * bi, bi), :].astype(jnp.bfloat16)
    a2 = Aa_ref[pl.ds(i * bi, bi), :].astype(jnp.bfloat16)

    da = hv_ref.shape[1]
    acc = jnp.zeros((bi, da), jnp.float32)
    rs = jnp.zeros((bi, 1), jnp.float32)
    for q in range(NQ):
        b1 = B_ref[:, q * BQ:(q + 1) * BQ].astype(jnp.bfloat16)
        b2 = Ba_ref[:, q * BQ:(q + 1) * BQ].astype(jnp.bfloat16)
        e = jnp.minimum(a1 * b1, a2 * b2)
        e = jnp.where(adj_refs[q][...] > 0, e, jnp.bfloat16(0))
        hv = hv_ref[q * BQ:(q + 1) * BQ, :]
        acc = acc + jnp.dot(e, hv, preferred_element_type=jnp.float32)
        if not fold_rs:
            rs = rs + jnp.sum(e.astype(jnp.float32), axis=1, keepdims=True)

    if fold_rs:
        rs = acc[:, d:d + 1]
    hp = acc[:, :d] / (rs + 1e-10)
    out_ref[...] = jnp.maximum(hp, 0.0)


def _gat_layer(adj, hv_b16, d, A, Aa, B, Ba, bi, fold_rs):
    ni = N // bi
    da = hv_b16.shape[1]
    adj_specs = tuple(
        pl.BlockSpec((bi, BQ), functools.partial(lambda q, i: (i, q), q))
        for q in range(NQ))
    return pl.pallas_call(
        functools.partial(_gat_kernel, bi, d, fold_rs),
        grid=(ni,),
        in_specs=(
            pl.BlockSpec((N, 1), lambda i: (0, 0)),
            pl.BlockSpec((N, 1), lambda i: (0, 0)),
            pl.BlockSpec((1, N), lambda i: (0, 0)),
            pl.BlockSpec((1, N), lambda i: (0, 0)),
            pl.BlockSpec((N, da), lambda i: (0, 0)),
        ) + adj_specs,
        out_specs=pl.BlockSpec((bi, d), lambda i: (i, 0)),
        out_shape=jax.ShapeDtypeStruct((N, d), jnp.float32),
        compiler_params=pltpu.CompilerParams(
            dimension_semantics=("arbitrary",)),
    )(A, Aa, B, Ba, hv_b16, *((adj,) * NQ))


def _pad_kernel(h2_ref, out_ref):
    out_ref[pl.ds(0, N), :] = h2_ref[...]
    out_ref[pl.ds(N, PAD_N - N), :] = jnp.zeros((PAD_N - N, h2_ref.shape[1]),
                                                jnp.float32)


def _pad_output(h2):
    d = h2.shape[1]
    return pl.pallas_call(
        _pad_kernel,
        out_shape=jax.ShapeDtypeStruct((PAD_N, d), jnp.float32),
    )(h2)


def kernel(x, adj, pad_n, pos_idx, W1, a1, W2, a2):
    hv1, A1, Aa1, B1, Ba1 = _prologue(x, W1, a1)
    d1 = hv1.shape[1]
    hv1_aug = jnp.concatenate(
        [hv1, jnp.ones((N, 1), jnp.float32),
         jnp.zeros((N, 7), jnp.float32)], axis=1).astype(jnp.bfloat16)
    h1 = _gat_layer(adj, hv1_aug, d1, A1, Aa1, B1, Ba1, bi=512, fold_rs=True)
    hv2, A2, Aa2, B2, Ba2 = _prologue(h1, W2, a2)
    d2 = hv2.shape[1]
    hv2_aug = jnp.concatenate(
        [hv2, jnp.ones((N, 1), jnp.float32),
         jnp.zeros((N, 127 - d2), jnp.float32)], axis=1).astype(jnp.bfloat16)
    h2 = _gat_layer(adj, hv2_aug, d2, A2, Aa2, B2, Ba2, bi=512, fold_rs=True)
    return _pad_output(h2)
